# Initial kernel scaffold; baseline (speedup 1.0000x reference)
import jax
import jax.numpy as jnp
from jax.experimental import pallas as pl


def kernel(x, token_table, pos_table, gamma, beta):
    raise NotImplementedError("write your pallas kernel here")


# speedup vs baseline: 13.6226x; 13.6226x over previous
import functools

import jax
import jax.numpy as jnp
from jax.experimental import pallas as pl

VOCAB = 5
EMBED_DIM = 256
MAX_LEN = 512
EPS = 1e-5

BATCH_BLK = 8


def _combine_kernel(tok_ref, pos_ref, gamma_ref, beta_ref, out_ref):
    emb = tok_ref[...][:, None, :] + pos_ref[...][None, :, :]
    mean = jnp.mean(emb, axis=-1, keepdims=True)
    var = jnp.mean(jnp.square(emb - mean), axis=-1, keepdims=True)
    normed = (emb - mean) * jax.lax.rsqrt(var + EPS)
    out_ref[...] = normed * gamma_ref[...][None, None, :] + beta_ref[...][None, None, :]


def _expand_kernel(x_ref, comb_ref, out_ref):
    xb = x_ref[...]
    c = comb_ref[...]
    sel = xb[:, :, None]
    r = jnp.where(sel == 0, c[0][None], c[4][None])
    r = jnp.where(sel == 1, c[1][None], r)
    r = jnp.where(sel == 2, c[2][None], r)
    r = jnp.where(sel == 3, c[3][None], r)
    out_ref[...] = r


@functools.partial(jax.jit, static_argnums=())
def kernel(x, token_table, pos_table, gamma, beta):
    batch, seq = x.shape
    vocab, dim = token_table.shape

    combined = pl.pallas_call(
        _combine_kernel,
        out_shape=jax.ShapeDtypeStruct((vocab, seq, dim), jnp.float32),
    )(token_table, pos_table[:seq], gamma, beta)

    x = x.astype(jnp.int32)
    grid = (batch // BATCH_BLK,)
    out = pl.pallas_call(
        _expand_kernel,
        grid=grid,
        in_specs=[
            pl.BlockSpec((BATCH_BLK, seq), lambda i: (i, 0)),
            pl.BlockSpec((vocab, seq, dim), lambda i: (0, 0, 0)),
        ],
        out_specs=pl.BlockSpec((BATCH_BLK, seq, dim), lambda i: (i, 0, 0)),
        out_shape=jax.ShapeDtypeStruct((batch, seq, dim), jnp.float32),
    )(x, combined)
    return out


# speedup vs baseline: 15.2786x; 1.1216x over previous
import functools

import jax
import jax.numpy as jnp
from jax.experimental import pallas as pl

VOCAB = 5
EMBED_DIM = 256
MAX_LEN = 512
EPS = 1e-5

BATCH_BLK = 16


def _combine_kernel(tok_ref, pos_ref, gamma_ref, beta_ref, out_ref):
    emb = tok_ref[...][:, None, :] + pos_ref[...][None, :, :]
    mean = jnp.mean(emb, axis=-1, keepdims=True)
    var = jnp.mean(jnp.square(emb - mean), axis=-1, keepdims=True)
    normed = (emb - mean) * jax.lax.rsqrt(var + EPS)
    out_ref[...] = normed * gamma_ref[...][None, None, :] + beta_ref[...][None, None, :]


def _expand_kernel(x_ref, comb_ref, out_ref):
    xb = x_ref[...]
    c = comb_ref[...]
    sel = xb[:, :, None]
    r = jnp.where(sel == 0, c[0][None], c[4][None])
    r = jnp.where(sel == 1, c[1][None], r)
    r = jnp.where(sel == 2, c[2][None], r)
    r = jnp.where(sel == 3, c[3][None], r)
    out_ref[...] = r


@functools.partial(jax.jit, static_argnums=())
def kernel(x, token_table, pos_table, gamma, beta):
    batch, seq = x.shape
    vocab, dim = token_table.shape

    combined = pl.pallas_call(
        _combine_kernel,
        out_shape=jax.ShapeDtypeStruct((vocab, seq, dim), jnp.float32),
    )(token_table, pos_table[:seq], gamma, beta)

    x = x.astype(jnp.int32)
    grid = (batch // BATCH_BLK,)
    out = pl.pallas_call(
        _expand_kernel,
        grid=grid,
        in_specs=[
            pl.BlockSpec((BATCH_BLK, seq), lambda i: (i, 0)),
            pl.BlockSpec((vocab, seq, dim), lambda i: (0, 0, 0)),
        ],
        out_specs=pl.BlockSpec((BATCH_BLK, seq, dim), lambda i: (i, 0, 0)),
        out_shape=jax.ShapeDtypeStruct((batch, seq, dim), jnp.float32),
    )(x, combined)
    return out


# speedup vs baseline: 15.6082x; 1.0216x over previous
import functools

import jax
import jax.numpy as jnp
from jax.experimental import pallas as pl

VOCAB = 5
EMBED_DIM = 256
MAX_LEN = 512
EPS = 1e-5

BATCH_BLK = 32


def _combine_kernel(tok_ref, pos_ref, gamma_ref, beta_ref, out_ref):
    emb = tok_ref[...][:, None, :] + pos_ref[...][None, :, :]
    mean = jnp.mean(emb, axis=-1, keepdims=True)
    var = jnp.mean(jnp.square(emb - mean), axis=-1, keepdims=True)
    normed = (emb - mean) * jax.lax.rsqrt(var + EPS)
    out_ref[...] = normed * gamma_ref[...][None, None, :] + beta_ref[...][None, None, :]


def _expand_kernel(x_ref, comb_ref, out_ref):
    xb = x_ref[...]
    c = comb_ref[...]
    sel = xb[:, :, None]
    r = jnp.where(sel == 0, c[0][None], c[4][None])
    r = jnp.where(sel == 1, c[1][None], r)
    r = jnp.where(sel == 2, c[2][None], r)
    r = jnp.where(sel == 3, c[3][None], r)
    out_ref[...] = r


@functools.partial(jax.jit, static_argnums=())
def kernel(x, token_table, pos_table, gamma, beta):
    batch, seq = x.shape
    vocab, dim = token_table.shape

    combined = pl.pallas_call(
        _combine_kernel,
        out_shape=jax.ShapeDtypeStruct((vocab, seq, dim), jnp.float32),
    )(token_table, pos_table[:seq], gamma, beta)

    x = x.astype(jnp.int32)
    grid = (batch // BATCH_BLK,)
    out = pl.pallas_call(
        _expand_kernel,
        grid=grid,
        in_specs=[
            pl.BlockSpec((BATCH_BLK, seq), lambda i: (i, 0)),
            pl.BlockSpec((vocab, seq, dim), lambda i: (0, 0, 0)),
        ],
        out_specs=pl.BlockSpec((BATCH_BLK, seq, dim), lambda i: (i, 0, 0)),
        out_shape=jax.ShapeDtypeStruct((batch, seq, dim), jnp.float32),
    )(x, combined)
    return out
